# baseline (speedup 1.0000x reference)
import functools

import jax
import jax.numpy as jnp
from jax import lax
from jax.experimental import pallas as pl
from jax.experimental.pallas import tpu as pltpu
from jax.experimental.pallas import tpu_sc as plsc

N, E, D, H = 10000, 160000, 256, 256
HALF = H // 2
NC, NS = 2, 16
L = 16
CHUNK = 64
BCH = 16
BE = BCH * CHUNK
NBLK = 160
EP = NBLK * BE
S0, S1 = 10, 0
RPS = 640
NPAD = NS * RPS

_mesh = plsc.VectorSubcoreMesh(core_axis_name="c", subcore_axis_name="s")


def _zero_acc(rows, acc, base):
    zeros16 = jnp.zeros((L,), jnp.float32)

    @pl.loop(0, CHUNK)
    def _(r):
        @pl.loop(0, HALF // L)
        def _(j):
            rows[3, r, pl.ds(j * L, L)] = zeros16

    for k in range(RPS // CHUNK):
        pltpu.sync_copy(rows.at[3], acc.at[pl.ds(base + k * CHUNK, CHUNK)])


def _sc_agg_body(compute_deg, y2, srcr, dst2, *rest):
    if compute_deg:
        out, deg, idxg_v, dst2_v, rows, acc, sga, sgb, ssa, ssb = rest
    else:
        out, idxg_v, dst2_v, rows, acc, sga, sgb, ssa, ssb = rest

    cid = lax.axis_index("c")
    sid = lax.axis_index("s")
    base = sid * RPS
    nblk = jnp.where(cid == 0, S0, S1)
    blk0 = jnp.where(cid == 0, sid * S0, NS * S0 + sid * S1)

    def gath(c, b, sem):
        idx = idxg_v.at[c // 2, pl.ds((c % 2) * CHUNK, CHUNK)]
        pltpu.async_copy(y2.at[idx], rows.at[b], sem)

    def gath_wait(b, sem):
        pltpu.make_async_copy(y2.at[pl.ds(0, CHUNK)], rows.at[b], sem).wait()

    def scat(c, b, sem):
        pltpu.async_copy(rows.at[b], acc.at[dst2_v.at[c]], sem, add=True)

    def scat_wait(b, sem):
        pltpu.make_async_copy(rows.at[b], acc.at[pl.ds(0, CHUNK)], sem).wait()

    for h in (0, 1):
        _zero_acc(rows, acc, base)
        plsc.subcore_barrier()

        @pl.loop(0, nblk)
        def _(k):
            blk = blk0 + k
            pltpu.sync_copy(srcr.at[blk], idxg_v)
            pltpu.sync_copy(dst2.at[blk], dst2_v)

            @pl.loop(0, BE // 128)
            def _(r):
                @pl.loop(0, 128 // L)
                def _(j):
                    s = idxg_v[r, pl.ds(j * L, L)]
                    idxg_v[r, pl.ds(j * L, L)] = s * 2 + h

            gath(0, 0, sga)
            gath(1, 1, sga)

            @pl.loop(0, BCH, step=4)
            def _(c):
                gath(c + 2, 2, sgb)
                gath(c + 3, 3, sgb)
                gath_wait(0, sga)
                gath_wait(1, sga)
                scat(c, 0, ssa)
                scat(c + 1, 1, ssa)
                scat_wait(0, ssa)
                scat_wait(1, ssa)

                @pl.when(c + 4 < BCH)
                def _():
                    gath(c + 4, 0, sga)
                    gath(c + 5, 1, sga)

                gath_wait(2, sgb)
                gath_wait(3, sgb)
                scat(c + 2, 2, ssb)
                scat(c + 3, 3, ssb)
                scat_wait(2, ssb)
                scat_wait(3, ssb)

        plsc.subcore_barrier()
        pltpu.sync_copy(acc.at[pl.ds(base, RPS)], out.at[cid, h, pl.ds(base, RPS)])

    if compute_deg:
        _zero_acc(rows, acc, base)
        ones16 = jnp.ones((L,), jnp.float32)

        @pl.loop(0, CHUNK)
        def _(r):
            @pl.loop(0, HALF // L)
            def _(j):
                rows[0, r, pl.ds(j * L, L)] = ones16

        plsc.subcore_barrier()

        @pl.loop(0, nblk)
        def _(k):
            pltpu.sync_copy(dst2.at[blk0 + k], dst2_v)

            @pl.loop(0, BCH, step=4)
            def _(c):
                for b in range(4):
                    scat(c + b, 0, ssa)
                for b in range(4):
                    scat_wait(0, ssa)

        plsc.subcore_barrier()
        pltpu.sync_copy(acc.at[pl.ds(base, RPS)], deg.at[cid, pl.ds(base, RPS)])


def _make_sc_agg(compute_deg):
    out_type = jax.ShapeDtypeStruct((NC, 2, NPAD, HALF), jnp.float32)
    if compute_deg:
        out_type = [out_type, jax.ShapeDtypeStruct((NC, NPAD, HALF), jnp.float32)]
    return pl.kernel(
        functools.partial(_sc_agg_body, compute_deg),
        out_type=out_type,
        mesh=_mesh,
        scratch_types=[
            pltpu.VMEM((BE // 128, 128), jnp.int32),
            pltpu.VMEM((BCH, CHUNK), jnp.int32),
            pltpu.VMEM((4, CHUNK, HALF), jnp.float32),
            pltpu.VMEM_SHARED((NPAD, HALF), jnp.float32),
            pltpu.SemaphoreType.DMA,
            pltpu.SemaphoreType.DMA,
            pltpu.SemaphoreType.DMA,
            pltpu.SemaphoreType.DMA,
        ],
        compiler_params=pltpu.CompilerParams(needs_layout_passes=False),
        name=f"sage_sc_agg_deg{int(compute_deg)}",
    )


_sc_agg_deg = _make_sc_agg(True)
_sc_agg = _make_sc_agg(False)

BN = 2000


def _mm_body(x_ref, ws_ref, wn_ref, ys_ref, yn_ref):
    x = x_ref[...]
    ys_ref[...] = jnp.dot(x, ws_ref[...], preferred_element_type=jnp.float32)
    yn_ref[...] = jnp.dot(x, wn_ref[...], preferred_element_type=jnp.float32)


_mm = pl.pallas_call(
    _mm_body,
    grid=(N // BN,),
    in_specs=[
        pl.BlockSpec((BN, D), lambda i: (i, 0)),
        pl.BlockSpec((D, H), lambda i: (0, 0)),
        pl.BlockSpec((D, H), lambda i: (0, 0)),
    ],
    out_specs=[pl.BlockSpec((BN, H), lambda i: (i, 0))] * 2,
    out_shape=[jax.ShapeDtypeStruct((N, H), jnp.float32)] * 2,
)


def _comb_core(ys_ref, a_ref, deg_ref, b_ref):
    a = a_ref[...]
    mean = jnp.concatenate([a[0, 0] + a[1, 0], a[0, 1] + a[1, 1]], axis=-1)
    recip = 1.0 / jnp.maximum(deg_ref[...], 1.0)
    return ys_ref[...] + mean * recip + b_ref[...]


def _comb_mm_body(with_scale, *refs):
    if with_scale:
        ys_ref, a_ref, deg_ref, b_ref, scale_ref, ws_ref, wn_ref, ys_o, yn_o = refs
    else:
        ys_ref, a_ref, deg_ref, b_ref, ws_ref, wn_ref, ys_o, yn_o = refs
    hv = jnp.tanh(_comb_core(ys_ref, a_ref, deg_ref, b_ref))
    if with_scale:
        hv = hv * scale_ref[...]
    ys_o[...] = jnp.dot(hv, ws_ref[...], preferred_element_type=jnp.float32)
    yn_o[...] = jnp.dot(hv, wn_ref[...], preferred_element_type=jnp.float32)


def _comb_final_body(ys_ref, a_ref, deg_ref, b_ref, out_ref):
    out_ref[...] = _comb_core(ys_ref, a_ref, deg_ref, b_ref)


_COMB_IN_SPECS = [
    pl.BlockSpec((BN, H), lambda i: (i, 0)),
    pl.BlockSpec((NC, 2, BN, HALF), lambda i: (0, 0, i, 0)),
    pl.BlockSpec((BN, 1), lambda i: (i, 0)),
    pl.BlockSpec((H,), lambda i: (0,)),
]


def _make_comb_mm(with_scale):
    extra = [pl.BlockSpec((BN, H), lambda i: (i, 0))] if with_scale else []
    return pl.pallas_call(
        functools.partial(_comb_mm_body, with_scale),
        grid=(N // BN,),
        in_specs=_COMB_IN_SPECS + extra + [
            pl.BlockSpec((H, H), lambda i: (0, 0)),
            pl.BlockSpec((H, H), lambda i: (0, 0)),
        ],
        out_specs=[pl.BlockSpec((BN, H), lambda i: (i, 0))] * 2,
        out_shape=[jax.ShapeDtypeStruct((N, H), jnp.float32)] * 2,
    )


_comb_mm_drop = _make_comb_mm(True)
_comb_mm = _make_comb_mm(False)

_comb_final = pl.pallas_call(
    _comb_final_body,
    grid=(N // BN,),
    in_specs=_COMB_IN_SPECS,
    out_specs=pl.BlockSpec((BN, H), lambda i: (i, 0)),
    out_shape=jax.ShapeDtypeStruct((N, H), jnp.float32),
)


def kernel(in_feat, edge_index, Ws1, Wn1, b1, Ws2, Wn2, b2, Ws3, Wn3, b3):
    src = edge_index[0]
    dst = edge_index[1]
    pad = EP - E
    src_p = jnp.concatenate([src, jnp.zeros((pad,), jnp.int32)])
    dst_p = jnp.concatenate([dst, jnp.full((pad,), N, jnp.int32)])
    srcr = src_p.reshape(NBLK, BE // 128, 128)
    dst2 = dst_p.reshape(NBLK, BCH, CHUNK)

    mask = jax.random.bernoulli(jax.random.key(42), 0.5, (N, H))
    scale = mask.astype(jnp.float32) * 2.0

    y1s, y1n = _mm(in_feat, Ws1, Wn1)
    agg1, degp = _sc_agg_deg(y1n.reshape(2 * N, HALF), srcr, dst2)
    deg = degp[0, :N, :1] + degp[1, :N, :1]
    y2s, y2n = _comb_mm_drop(y1s, agg1, deg, b1, scale, Ws2, Wn2)
    agg2 = _sc_agg(y2n.reshape(2 * N, HALF), srcr, dst2)
    y3s, y3n = _comb_mm(y2s, agg2, deg, b2, Ws3, Wn3)
    agg3 = _sc_agg(y3n.reshape(2 * N, HALF), srcr, dst2)
    return _comb_final(y3s, agg3, deg, b3)


# speedup vs baseline: 3.2788x; 3.2788x over previous
import functools

import jax
import jax.numpy as jnp
from jax import lax
from jax.experimental import pallas as pl
from jax.experimental.pallas import tpu as pltpu
from jax.experimental.pallas import tpu_sc as plsc

N, E, D, H = 10000, 160000, 256, 256
HALF = H // 2
NC, NS = 2, 16
L = 16
CHUNK = 64
BCH = 16
BE = BCH * CHUNK
NBLK = 160
EP = NBLK * BE
S0, S1 = 5, 5
RPS = 640
NPAD = NS * RPS

_mesh = plsc.VectorSubcoreMesh(core_axis_name="c", subcore_axis_name="s")


def _zero_acc(rows, acc, base):
    zeros16 = jnp.zeros((L,), jnp.float32)

    @pl.loop(0, CHUNK)
    def _(r):
        @pl.loop(0, HALF // L)
        def _(j):
            rows[3, r, pl.ds(j * L, L)] = zeros16

    for k in range(RPS // CHUNK):
        pltpu.sync_copy(rows.at[3], acc.at[pl.ds(base + k * CHUNK, CHUNK)])


def _sc_agg_body(compute_deg, y2, srcr, dst2, *rest):
    if compute_deg:
        out, deg, idxg_v, dst2_v, rows, acc, sga, sgb, ssa, ssb = rest
    else:
        out, idxg_v, dst2_v, rows, acc, sga, sgb, ssa, ssb = rest

    cid = lax.axis_index("c")
    sid = lax.axis_index("s")
    base = sid * RPS
    nblk = jnp.where(cid == 0, S0, S1)
    blk0 = jnp.where(cid == 0, sid * S0, NS * S0 + sid * S1)

    def gath(c, b, sem):
        idx = idxg_v.at[c // 2, pl.ds((c % 2) * CHUNK, CHUNK)]
        pltpu.async_copy(y2.at[idx], rows.at[b], sem)

    def gath_wait(b, sem):
        pltpu.make_async_copy(y2.at[pl.ds(0, CHUNK)], rows.at[b], sem).wait()

    def scat(c, b, sem):
        pltpu.async_copy(rows.at[b], acc.at[dst2_v.at[c]], sem, add=True)

    def scat_wait(b, sem):
        pltpu.make_async_copy(rows.at[b], acc.at[pl.ds(0, CHUNK)], sem).wait()

    for h in (0, 1):
        _zero_acc(rows, acc, base)
        plsc.subcore_barrier()

        @pl.loop(0, nblk)
        def _(k):
            blk = blk0 + k
            pltpu.sync_copy(srcr.at[blk], idxg_v)
            pltpu.sync_copy(dst2.at[blk], dst2_v)

            @pl.loop(0, BE // 128)
            def _(r):
                @pl.loop(0, 128 // L)
                def _(j):
                    s = idxg_v[r, pl.ds(j * L, L)]
                    idxg_v[r, pl.ds(j * L, L)] = s * 2 + h

            gath(0, 0, sga)
            gath(1, 1, sga)

            @pl.loop(0, BCH, step=4)
            def _(c):
                gath(c + 2, 2, sgb)
                gath(c + 3, 3, sgb)
                gath_wait(0, sga)
                gath_wait(1, sga)
                scat(c, 0, ssa)
                scat(c + 1, 1, ssa)
                scat_wait(0, ssa)
                scat_wait(1, ssa)

                @pl.when(c + 4 < BCH)
                def _():
                    gath(c + 4, 0, sga)
                    gath(c + 5, 1, sga)

                gath_wait(2, sgb)
                gath_wait(3, sgb)
                scat(c + 2, 2, ssb)
                scat(c + 3, 3, ssb)
                scat_wait(2, ssb)
                scat_wait(3, ssb)

        plsc.subcore_barrier()
        pltpu.sync_copy(acc.at[pl.ds(base, RPS)], out.at[cid, h, pl.ds(base, RPS)])

    if compute_deg:
        _zero_acc(rows, acc, base)
        ones16 = jnp.ones((L,), jnp.float32)

        @pl.loop(0, CHUNK)
        def _(r):
            @pl.loop(0, HALF // L)
            def _(j):
                rows[0, r, pl.ds(j * L, L)] = ones16

        plsc.subcore_barrier()

        @pl.loop(0, nblk)
        def _(k):
            pltpu.sync_copy(dst2.at[blk0 + k], dst2_v)

            @pl.loop(0, BCH, step=4)
            def _(c):
                for b in range(4):
                    scat(c + b, 0, ssa)
                for b in range(4):
                    scat_wait(0, ssa)

        plsc.subcore_barrier()
        pltpu.sync_copy(acc.at[pl.ds(base, RPS)], deg.at[cid, pl.ds(base, RPS)])


def _make_sc_agg(compute_deg):
    out_type = jax.ShapeDtypeStruct((NC, 2, NPAD, HALF), jnp.float32)
    if compute_deg:
        out_type = [out_type, jax.ShapeDtypeStruct((NC, NPAD, HALF), jnp.float32)]
    return pl.kernel(
        functools.partial(_sc_agg_body, compute_deg),
        out_type=out_type,
        mesh=_mesh,
        scratch_types=[
            pltpu.VMEM((BE // 128, 128), jnp.int32),
            pltpu.VMEM((BCH, CHUNK), jnp.int32),
            pltpu.VMEM((4, CHUNK, HALF), jnp.float32),
            pltpu.VMEM_SHARED((NPAD, HALF), jnp.float32),
            pltpu.SemaphoreType.DMA,
            pltpu.SemaphoreType.DMA,
            pltpu.SemaphoreType.DMA,
            pltpu.SemaphoreType.DMA,
        ],
        compiler_params=pltpu.CompilerParams(needs_layout_passes=False),
        name=f"sage_sc_agg_deg{int(compute_deg)}",
    )


_sc_agg_deg = _make_sc_agg(True)
_sc_agg = _make_sc_agg(False)

BN = 2000


def _mm_body(x_ref, ws_ref, wn_ref, ys_ref, yn_ref):
    x = x_ref[...]
    ys_ref[...] = jnp.dot(x, ws_ref[...], preferred_element_type=jnp.float32)
    yn_ref[...] = jnp.dot(x, wn_ref[...], preferred_element_type=jnp.float32)


_mm = pl.pallas_call(
    _mm_body,
    grid=(N // BN,),
    in_specs=[
        pl.BlockSpec((BN, D), lambda i: (i, 0)),
        pl.BlockSpec((D, H), lambda i: (0, 0)),
        pl.BlockSpec((D, H), lambda i: (0, 0)),
    ],
    out_specs=[pl.BlockSpec((BN, H), lambda i: (i, 0))] * 2,
    out_shape=[jax.ShapeDtypeStruct((N, H), jnp.float32)] * 2,
)


def _comb_core(ys_ref, a_ref, deg_ref, b_ref):
    a = a_ref[...]
    mean = jnp.concatenate([a[0, 0] + a[1, 0], a[0, 1] + a[1, 1]], axis=-1)
    recip = 1.0 / jnp.maximum(deg_ref[...], 1.0)
    return ys_ref[...] + mean * recip + b_ref[...]


def _comb_mm_body(with_scale, *refs):
    if with_scale:
        ys_ref, a_ref, deg_ref, b_ref, scale_ref, ws_ref, wn_ref, ys_o, yn_o = refs
    else:
        ys_ref, a_ref, deg_ref, b_ref, ws_ref, wn_ref, ys_o, yn_o = refs
    hv = jnp.tanh(_comb_core(ys_ref, a_ref, deg_ref, b_ref))
    if with_scale:
        hv = hv * scale_ref[...]
    ys_o[...] = jnp.dot(hv, ws_ref[...], preferred_element_type=jnp.float32)
    yn_o[...] = jnp.dot(hv, wn_ref[...], preferred_element_type=jnp.float32)


def _comb_final_body(ys_ref, a_ref, deg_ref, b_ref, out_ref):
    out_ref[...] = _comb_core(ys_ref, a_ref, deg_ref, b_ref)


_COMB_IN_SPECS = [
    pl.BlockSpec((BN, H), lambda i: (i, 0)),
    pl.BlockSpec((NC, 2, BN, HALF), lambda i: (0, 0, i, 0)),
    pl.BlockSpec((BN, 1), lambda i: (i, 0)),
    pl.BlockSpec((H,), lambda i: (0,)),
]


def _make_comb_mm(with_scale):
    extra = [pl.BlockSpec((BN, H), lambda i: (i, 0))] if with_scale else []
    return pl.pallas_call(
        functools.partial(_comb_mm_body, with_scale),
        grid=(N // BN,),
        in_specs=_COMB_IN_SPECS + extra + [
            pl.BlockSpec((H, H), lambda i: (0, 0)),
            pl.BlockSpec((H, H), lambda i: (0, 0)),
        ],
        out_specs=[pl.BlockSpec((BN, H), lambda i: (i, 0))] * 2,
        out_shape=[jax.ShapeDtypeStruct((N, H), jnp.float32)] * 2,
    )


_comb_mm_drop = _make_comb_mm(True)
_comb_mm = _make_comb_mm(False)

_comb_final = pl.pallas_call(
    _comb_final_body,
    grid=(N // BN,),
    in_specs=_COMB_IN_SPECS,
    out_specs=pl.BlockSpec((BN, H), lambda i: (i, 0)),
    out_shape=jax.ShapeDtypeStruct((N, H), jnp.float32),
)


def kernel(in_feat, edge_index, Ws1, Wn1, b1, Ws2, Wn2, b2, Ws3, Wn3, b3):
    src = edge_index[0]
    dst = edge_index[1]
    pad = EP - E
    spread = jnp.arange(pad, dtype=jnp.int32)
    src_p = jnp.concatenate([src, spread % N])
    dst_p = jnp.concatenate([dst, N + spread % (NPAD - N)])
    srcr = src_p.reshape(NBLK, BE // 128, 128)
    dst2 = dst_p.reshape(NBLK, BCH, CHUNK)

    mask = jax.random.bernoulli(jax.random.key(42), 0.5, (N, H))
    scale = mask.astype(jnp.float32) * 2.0

    y1s, y1n = _mm(in_feat, Ws1, Wn1)
    agg1, degp = _sc_agg_deg(y1n.reshape(2 * N, HALF), srcr, dst2)
    deg = degp[0, :N, :1] + degp[1, :N, :1]
    y2s, y2n = _comb_mm_drop(y1s, agg1, deg, b1, scale, Ws2, Wn2)
    agg2 = _sc_agg(y2n.reshape(2 * N, HALF), srcr, dst2)
    y3s, y3n = _comb_mm(y2s, agg2, deg, b2, Ws3, Wn3)
    agg3 = _sc_agg(y3n.reshape(2 * N, HALF), srcr, dst2)
    return _comb_final(y3s, agg3, deg, b3)


# speedup vs baseline: 4.0557x; 1.2370x over previous
import functools

import jax
import jax.numpy as jnp
from jax import lax
from jax.experimental import pallas as pl
from jax.experimental.pallas import tpu as pltpu
from jax.experimental.pallas import tpu_sc as plsc

N, E, D, H = 10000, 160000, 256, 256
HALF = H // 2
NC, NS = 2, 16
NW = NC * NS
L = 16
CHUNK = 64
NCH = 80
EWP = NCH * CHUNK
EP = NW * EWP
RPS = 640
NPAD = NS * RPS

_mesh = plsc.VectorSubcoreMesh(core_axis_name="c", subcore_axis_name="s")


def _zero_acc(rows, acc, base):
    zeros16 = jnp.zeros((L,), jnp.float32)

    @pl.loop(0, CHUNK)
    def _(r):
        @pl.loop(0, HALF // L)
        def _(j):
            rows[3, r, pl.ds(j * L, L)] = zeros16

    for k in range(RPS // CHUNK):
        pltpu.sync_copy(rows.at[3], acc.at[pl.ds(base + k * CHUNK, CHUNK)])


def _sc_agg_body(compute_deg, ylo, yhi, srcr, dst2, *rest):
    if compute_deg:
        out, deg, src_v, dst2_v, rows, acc, sga, sgb, ssa, ssb = rest
    else:
        out, src_v, dst2_v, rows, acc, sga, sgb, ssa, ssb = rest

    cid = lax.axis_index("c")
    sid = lax.axis_index("s")
    wid = sid * NC + cid
    base = sid * RPS

    pltpu.sync_copy(srcr.at[wid], src_v)
    pltpu.sync_copy(dst2.at[wid], dst2_v)

    def gath(tab, c, b, sem):
        idx = src_v.at[c // 2, pl.ds((c % 2) * CHUNK, CHUNK)]
        pltpu.async_copy(tab.at[idx], rows.at[b], sem)

    def gath_wait(b, sem):
        pltpu.make_async_copy(ylo.at[pl.ds(0, CHUNK)], rows.at[b], sem).wait()

    def scat(c, b, sem):
        pltpu.async_copy(rows.at[b], acc.at[dst2_v.at[c]], sem, add=True)

    def scat_wait(b, sem):
        pltpu.make_async_copy(rows.at[b], acc.at[pl.ds(0, CHUNK)], sem).wait()

    for h, tab in ((0, ylo), (1, yhi)):
        gath(tab, 0, 0, sga)
        gath(tab, 1, 1, sga)
        _zero_acc(rows, acc, base)
        plsc.subcore_barrier()

        @pl.loop(0, NCH, step=4)
        def _(c):
            gath(tab, c + 2, 2, sgb)
            gath(tab, c + 3, 3, sgb)
            gath_wait(0, sga)
            gath_wait(1, sga)
            scat(c, 0, ssa)
            scat(c + 1, 1, ssa)
            scat_wait(0, ssa)
            scat_wait(1, ssa)

            @pl.when(c + 4 < NCH)
            def _():
                gath(tab, c + 4, 0, sga)
                gath(tab, c + 5, 1, sga)

            gath_wait(2, sgb)
            gath_wait(3, sgb)
            scat(c + 2, 2, ssb)
            scat(c + 3, 3, ssb)
            scat_wait(2, ssb)
            scat_wait(3, ssb)

        plsc.subcore_barrier()
        pltpu.sync_copy(acc.at[pl.ds(base, RPS)], out.at[cid, h, pl.ds(base, RPS)])

    if compute_deg:
        _zero_acc(rows, acc, base)
        ones16 = jnp.ones((L,), jnp.float32)

        @pl.loop(0, CHUNK)
        def _(r):
            @pl.loop(0, HALF // L)
            def _(j):
                rows[0, r, pl.ds(j * L, L)] = ones16

        plsc.subcore_barrier()

        @pl.loop(0, NCH, step=4)
        def _(c):
            for b in range(4):
                scat(c + b, 0, ssa)
            for b in range(4):
                scat_wait(0, ssa)

        plsc.subcore_barrier()
        pltpu.sync_copy(acc.at[pl.ds(base, RPS)], deg.at[cid, pl.ds(base, RPS)])


def _make_sc_agg(compute_deg):
    out_type = jax.ShapeDtypeStruct((NC, 2, NPAD, HALF), jnp.float32)
    if compute_deg:
        out_type = [out_type, jax.ShapeDtypeStruct((NC, NPAD, HALF), jnp.float32)]
    return pl.kernel(
        functools.partial(_sc_agg_body, compute_deg),
        out_type=out_type,
        mesh=_mesh,
        scratch_types=[
            pltpu.VMEM((EWP // 128, 128), jnp.int32),
            pltpu.VMEM((NCH, CHUNK), jnp.int32),
            pltpu.VMEM((4, CHUNK, HALF), jnp.float32),
            pltpu.VMEM_SHARED((NPAD, HALF), jnp.float32),
            pltpu.SemaphoreType.DMA,
            pltpu.SemaphoreType.DMA,
            pltpu.SemaphoreType.DMA,
            pltpu.SemaphoreType.DMA,
        ],
        compiler_params=pltpu.CompilerParams(needs_layout_passes=False),
        name=f"sage_sc_agg_deg{int(compute_deg)}",
    )


_sc_agg_deg = _make_sc_agg(True)
_sc_agg = _make_sc_agg(False)

BN = 2000


def _mm_body(x_ref, ws_ref, wn_ref, ys_ref, ylo_ref, yhi_ref):
    x = x_ref[...]
    ys_ref[...] = jnp.dot(x, ws_ref[...], preferred_element_type=jnp.float32)
    yn = jnp.dot(x, wn_ref[...], preferred_element_type=jnp.float32)
    ylo_ref[...] = yn[:, :HALF]
    yhi_ref[...] = yn[:, HALF:]


_MM_OUT_SPECS = [
    pl.BlockSpec((BN, H), lambda i: (i, 0)),
    pl.BlockSpec((BN, HALF), lambda i: (i, 0)),
    pl.BlockSpec((BN, HALF), lambda i: (i, 0)),
]
_MM_OUT_SHAPE = [
    jax.ShapeDtypeStruct((N, H), jnp.float32),
    jax.ShapeDtypeStruct((N, HALF), jnp.float32),
    jax.ShapeDtypeStruct((N, HALF), jnp.float32),
]

_mm = pl.pallas_call(
    _mm_body,
    grid=(N // BN,),
    in_specs=[
        pl.BlockSpec((BN, D), lambda i: (i, 0)),
        pl.BlockSpec((D, H), lambda i: (0, 0)),
        pl.BlockSpec((D, H), lambda i: (0, 0)),
    ],
    out_specs=_MM_OUT_SPECS,
    out_shape=_MM_OUT_SHAPE,
)


def _comb_core(ys_ref, a_ref, deg_ref, b_ref):
    a = a_ref[...]
    mean = jnp.concatenate([a[0, 0] + a[1, 0], a[0, 1] + a[1, 1]], axis=-1)
    recip = 1.0 / jnp.maximum(deg_ref[...], 1.0)
    return ys_ref[...] + mean * recip + b_ref[...]


def _comb_mm_body(with_scale, *refs):
    if with_scale:
        ys_ref, a_ref, deg_ref, b_ref, scale_ref, ws_ref, wn_ref, ys_o, ylo_o, yhi_o = refs
    else:
        ys_ref, a_ref, deg_ref, b_ref, ws_ref, wn_ref, ys_o, ylo_o, yhi_o = refs
    hv = jnp.tanh(_comb_core(ys_ref, a_ref, deg_ref, b_ref))
    if with_scale:
        hv = hv * scale_ref[...]
    ys_o[...] = jnp.dot(hv, ws_ref[...], preferred_element_type=jnp.float32)
    yn = jnp.dot(hv, wn_ref[...], preferred_element_type=jnp.float32)
    ylo_o[...] = yn[:, :HALF]
    yhi_o[...] = yn[:, HALF:]


def _comb_final_body(ys_ref, a_ref, deg_ref, b_ref, out_ref):
    out_ref[...] = _comb_core(ys_ref, a_ref, deg_ref, b_ref)


_COMB_IN_SPECS = [
    pl.BlockSpec((BN, H), lambda i: (i, 0)),
    pl.BlockSpec((NC, 2, BN, HALF), lambda i: (0, 0, i, 0)),
    pl.BlockSpec((BN, 1), lambda i: (i, 0)),
    pl.BlockSpec((H,), lambda i: (0,)),
]


def _make_comb_mm(with_scale):
    extra = [pl.BlockSpec((BN, H), lambda i: (i, 0))] if with_scale else []
    return pl.pallas_call(
        functools.partial(_comb_mm_body, with_scale),
        grid=(N // BN,),
        in_specs=_COMB_IN_SPECS + extra + [
            pl.BlockSpec((H, H), lambda i: (0, 0)),
            pl.BlockSpec((H, H), lambda i: (0, 0)),
        ],
        out_specs=_MM_OUT_SPECS,
        out_shape=_MM_OUT_SHAPE,
    )


_comb_mm_drop = _make_comb_mm(True)
_comb_mm = _make_comb_mm(False)

_comb_final = pl.pallas_call(
    _comb_final_body,
    grid=(N // BN,),
    in_specs=_COMB_IN_SPECS,
    out_specs=pl.BlockSpec((BN, H), lambda i: (i, 0)),
    out_shape=jax.ShapeDtypeStruct((N, H), jnp.float32),
)


def kernel(in_feat, edge_index, Ws1, Wn1, b1, Ws2, Wn2, b2, Ws3, Wn3, b3):
    src = edge_index[0]
    dst = edge_index[1]
    pad = EP - E
    spread = jnp.arange(pad, dtype=jnp.int32)
    src_p = jnp.concatenate([src, spread % N])
    dst_p = jnp.concatenate([dst, N + spread % (NPAD - N)])
    srcr = src_p.reshape(NW, EWP // 128, 128)
    dst2 = dst_p.reshape(NW, NCH, CHUNK)

    mask = jax.random.bernoulli(jax.random.key(42), 0.5, (N, H))
    scale = mask.astype(jnp.float32) * 2.0

    y1s, y1lo, y1hi = _mm(in_feat, Ws1, Wn1)
    agg1, degp = _sc_agg_deg(y1lo, y1hi, srcr, dst2)
    deg = degp[0, :N, :1] + degp[1, :N, :1]
    y2s, y2lo, y2hi = _comb_mm_drop(y1s, agg1, deg, b1, scale, Ws2, Wn2)
    agg2 = _sc_agg(y2lo, y2hi, srcr, dst2)
    y3s, y3lo, y3hi = _comb_mm(y2s, agg2, deg, b2, Ws3, Wn3)
    agg3 = _sc_agg(y3lo, y3hi, srcr, dst2)
    return _comb_final(y3s, agg3, deg, b3)


# speedup vs baseline: 4.0592x; 1.0009x over previous
import functools

import jax
import jax.numpy as jnp
from jax import lax
from jax.experimental import pallas as pl
from jax.experimental.pallas import tpu as pltpu
from jax.experimental.pallas import tpu_sc as plsc

N, E, D, H = 10000, 160000, 256, 256
HALF = H // 2
NC, NS = 2, 16
NW = NC * NS
L = 16
CHUNK = 64
NCH = 80
EWP = NCH * CHUNK
EP = NW * EWP
RPS = 640
NPAD = NS * RPS

_mesh = plsc.VectorSubcoreMesh(core_axis_name="c", subcore_axis_name="s")


def _zero_acc(rows, acc, base):
    zeros16 = jnp.zeros((L,), jnp.float32)

    @pl.loop(0, CHUNK)
    def _(r):
        @pl.loop(0, HALF // L)
        def _(j):
            rows[3, r, pl.ds(j * L, L)] = zeros16

    for k in range(RPS // CHUNK):
        pltpu.sync_copy(rows.at[3], acc.at[pl.ds(base + k * CHUNK, CHUNK)])


def _sc_agg_body(compute_deg, ylo, yhi, srcr, dst2, *rest):
    if compute_deg:
        out, deg, src_v, dst2_v, rows, acc, sga, sgb, ssa, ssb = rest
    else:
        out, src_v, dst2_v, rows, acc, sga, sgb, ssa, ssb = rest

    cid = lax.axis_index("c")
    sid = lax.axis_index("s")
    base = sid * RPS

    def gath(tab, c, b, sem):
        idx = src_v.at[c // 2, pl.ds((c % 2) * CHUNK, CHUNK)]
        pltpu.async_copy(tab.at[idx], rows.at[b], sem)

    def gath_wait(b, sem):
        pltpu.make_async_copy(ylo.at[pl.ds(0, CHUNK)], rows.at[b], sem).wait()

    def scat(c, b, sem):
        pltpu.async_copy(rows.at[b], acc.at[dst2_v.at[c]], sem, add=True)

    def scat_wait(b, sem):
        pltpu.make_async_copy(rows.at[b], acc.at[pl.ds(0, CHUNK)], sem).wait()

    def run_edges(tab):
        @pl.loop(0, 2)
        def _(seg):
            pltpu.sync_copy(srcr.at[seg, sid], src_v)
            pltpu.sync_copy(dst2.at[seg, sid], dst2_v)
            gath(tab, 0, 0, sga)
            gath(tab, 1, 1, sga)

            @pl.loop(0, NCH, step=4)
            def _(c):
                gath(tab, c + 2, 2, sgb)
                gath(tab, c + 3, 3, sgb)
                gath_wait(0, sga)
                gath_wait(1, sga)
                scat(c, 0, ssa)
                scat(c + 1, 1, ssa)
                scat_wait(0, ssa)
                scat_wait(1, ssa)

                @pl.when(c + 4 < NCH)
                def _():
                    gath(tab, c + 4, 0, sga)
                    gath(tab, c + 5, 1, sga)

                gath_wait(2, sgb)
                gath_wait(3, sgb)
                scat(c + 2, 2, ssb)
                scat(c + 3, 3, ssb)
                scat_wait(2, ssb)
                scat_wait(3, ssb)

    _zero_acc(rows, acc, base)
    plsc.subcore_barrier()

    @pl.when(cid == 0)
    def _():
        run_edges(ylo)

    @pl.when(cid == 1)
    def _():
        run_edges(yhi)

    plsc.subcore_barrier()
    pltpu.sync_copy(acc.at[pl.ds(base, RPS)], out.at[cid, pl.ds(base, RPS)])

    if compute_deg:
        _zero_acc(rows, acc, base)
        ones16 = jnp.ones((L,), jnp.float32)

        @pl.loop(0, CHUNK)
        def _(r):
            @pl.loop(0, HALF // L)
            def _(j):
                rows[0, r, pl.ds(j * L, L)] = ones16

        plsc.subcore_barrier()

        @pl.when(cid == 0)
        def _():
            @pl.loop(0, 2)
            def _(seg):
                pltpu.sync_copy(dst2.at[seg, sid], dst2_v)

                @pl.loop(0, NCH, step=4)
                def _(c):
                    for b in range(4):
                        scat(c + b, 0, ssa)
                    for b in range(4):
                        scat_wait(0, ssa)

            plsc.subcore_barrier()
            pltpu.sync_copy(acc.at[pl.ds(base, RPS)], deg.at[pl.ds(base, RPS)])


def _make_sc_agg(compute_deg):
    out_type = jax.ShapeDtypeStruct((NC, NPAD, HALF), jnp.float32)
    if compute_deg:
        out_type = [out_type, jax.ShapeDtypeStruct((NPAD, HALF), jnp.float32)]
    return pl.kernel(
        functools.partial(_sc_agg_body, compute_deg),
        out_type=out_type,
        mesh=_mesh,
        scratch_types=[
            pltpu.VMEM((EWP // 128, 128), jnp.int32),
            pltpu.VMEM((NCH, CHUNK), jnp.int32),
            pltpu.VMEM((4, CHUNK, HALF), jnp.float32),
            pltpu.VMEM_SHARED((NPAD, HALF), jnp.float32),
            pltpu.SemaphoreType.DMA,
            pltpu.SemaphoreType.DMA,
            pltpu.SemaphoreType.DMA,
            pltpu.SemaphoreType.DMA,
        ],
        compiler_params=pltpu.CompilerParams(needs_layout_passes=False),
        name=f"sage_sc_agg_deg{int(compute_deg)}",
    )


_sc_agg_deg = _make_sc_agg(True)
_sc_agg = _make_sc_agg(False)

BN = 2000


def _mm_body(x_ref, ws_ref, wn_ref, ys_ref, ylo_ref, yhi_ref):
    x = x_ref[...]
    ys_ref[...] = jnp.dot(x, ws_ref[...], preferred_element_type=jnp.float32)
    yn = jnp.dot(x, wn_ref[...], preferred_element_type=jnp.float32)
    ylo_ref[...] = yn[:, :HALF]
    yhi_ref[...] = yn[:, HALF:]


_MM_OUT_SPECS = [
    pl.BlockSpec((BN, H), lambda i: (i, 0)),
    pl.BlockSpec((BN, HALF), lambda i: (i, 0)),
    pl.BlockSpec((BN, HALF), lambda i: (i, 0)),
]
_MM_OUT_SHAPE = [
    jax.ShapeDtypeStruct((N, H), jnp.float32),
    jax.ShapeDtypeStruct((N, HALF), jnp.float32),
    jax.ShapeDtypeStruct((N, HALF), jnp.float32),
]

_mm = pl.pallas_call(
    _mm_body,
    grid=(N // BN,),
    in_specs=[
        pl.BlockSpec((BN, D), lambda i: (i, 0)),
        pl.BlockSpec((D, H), lambda i: (0, 0)),
        pl.BlockSpec((D, H), lambda i: (0, 0)),
    ],
    out_specs=_MM_OUT_SPECS,
    out_shape=_MM_OUT_SHAPE,
)


def _comb_core(ys_ref, a_ref, deg_ref, b_ref):
    a = a_ref[...]
    mean = jnp.concatenate([a[0], a[1]], axis=-1)
    recip = 1.0 / jnp.maximum(deg_ref[...], 1.0)
    return ys_ref[...] + mean * recip + b_ref[...]


def _comb_mm_body(with_scale, *refs):
    if with_scale:
        ys_ref, a_ref, deg_ref, b_ref, scale_ref, ws_ref, wn_ref, ys_o, ylo_o, yhi_o = refs
    else:
        ys_ref, a_ref, deg_ref, b_ref, ws_ref, wn_ref, ys_o, ylo_o, yhi_o = refs
    hv = jnp.tanh(_comb_core(ys_ref, a_ref, deg_ref, b_ref))
    if with_scale:
        hv = hv * scale_ref[...]
    ys_o[...] = jnp.dot(hv, ws_ref[...], preferred_element_type=jnp.float32)
    yn = jnp.dot(hv, wn_ref[...], preferred_element_type=jnp.float32)
    ylo_o[...] = yn[:, :HALF]
    yhi_o[...] = yn[:, HALF:]


def _comb_final_body(ys_ref, a_ref, deg_ref, b_ref, out_ref):
    out_ref[...] = _comb_core(ys_ref, a_ref, deg_ref, b_ref)


_COMB_IN_SPECS = [
    pl.BlockSpec((BN, H), lambda i: (i, 0)),
    pl.BlockSpec((NC, BN, HALF), lambda i: (0, i, 0)),
    pl.BlockSpec((BN, 1), lambda i: (i, 0)),
    pl.BlockSpec((H,), lambda i: (0,)),
]


def _make_comb_mm(with_scale):
    extra = [pl.BlockSpec((BN, H), lambda i: (i, 0))] if with_scale else []
    return pl.pallas_call(
        functools.partial(_comb_mm_body, with_scale),
        grid=(N // BN,),
        in_specs=_COMB_IN_SPECS + extra + [
            pl.BlockSpec((H, H), lambda i: (0, 0)),
            pl.BlockSpec((H, H), lambda i: (0, 0)),
        ],
        out_specs=_MM_OUT_SPECS,
        out_shape=_MM_OUT_SHAPE,
    )


_comb_mm_drop = _make_comb_mm(True)
_comb_mm = _make_comb_mm(False)

_comb_final = pl.pallas_call(
    _comb_final_body,
    grid=(N // BN,),
    in_specs=_COMB_IN_SPECS,
    out_specs=pl.BlockSpec((BN, H), lambda i: (i, 0)),
    out_shape=jax.ShapeDtypeStruct((N, H), jnp.float32),
)


def kernel(in_feat, edge_index, Ws1, Wn1, b1, Ws2, Wn2, b2, Ws3, Wn3, b3):
    src = edge_index[0]
    dst = edge_index[1]
    pad = EP - E
    spread = jnp.arange(pad, dtype=jnp.int32)
    src_p = jnp.concatenate([src, spread % N])
    dst_p = jnp.concatenate([dst, N + spread % (NPAD - N)])
    srcr = src_p.reshape(2, NS, EWP // 128, 128)
    dst2 = dst_p.reshape(2, NS, NCH, CHUNK)

    mask = jax.random.bernoulli(jax.random.key(42), 0.5, (N, H))
    scale = mask.astype(jnp.float32) * 2.0

    y1s, y1lo, y1hi = _mm(in_feat, Ws1, Wn1)
    agg1, degp = _sc_agg_deg(y1lo, y1hi, srcr, dst2)
    deg = degp[:N, :1]
    y2s, y2lo, y2hi = _comb_mm_drop(y1s, agg1, deg, b1, scale, Ws2, Wn2)
    agg2 = _sc_agg(y2lo, y2hi, srcr, dst2)
    y3s, y3lo, y3hi = _comb_mm(y2s, agg2, deg, b2, Ws3, Wn3)
    agg3 = _sc_agg(y3lo, y3hi, srcr, dst2)
    return _comb_final(y3s, agg3, deg, b3)


# speedup vs baseline: 4.0769x; 1.0044x over previous
import functools

import jax
import jax.numpy as jnp
from jax import lax
from jax.experimental import pallas as pl
from jax.experimental.pallas import tpu as pltpu
from jax.experimental.pallas import tpu_sc as plsc

N, E, D, H = 10000, 160000, 256, 256
HALF = H // 2
NC, NS = 2, 16
NW = NC * NS
L = 16
CHUNK = 128
NCH = 40
EWP = NCH * CHUNK
EP = NW * EWP
RPS = 640
NPAD = NS * RPS

_mesh = plsc.VectorSubcoreMesh(core_axis_name="c", subcore_axis_name="s")


def _zero_acc(rows, acc, base):
    zeros16 = jnp.zeros((L,), jnp.float32)

    @pl.loop(0, CHUNK)
    def _(r):
        @pl.loop(0, HALF // L)
        def _(j):
            rows[1, r, pl.ds(j * L, L)] = zeros16

    for k in range(RPS // CHUNK):
        pltpu.sync_copy(rows.at[1], acc.at[pl.ds(base + k * CHUNK, CHUNK)])


def _sc_agg_body(compute_deg, ylo, yhi, srcr, dst2, *rest):
    if compute_deg:
        out, deg, src_v, dst2_v, rows, acc, sga, sgb, ssa, ssb = rest
    else:
        out, src_v, dst2_v, rows, acc, sga, sgb, ssa, ssb = rest

    cid = lax.axis_index("c")
    sid = lax.axis_index("s")
    base = sid * RPS

    def gath(tab, c, b, sem):
        pltpu.async_copy(tab.at[src_v.at[c]], rows.at[b], sem)

    def gath_wait(b, sem):
        pltpu.make_async_copy(ylo.at[pl.ds(0, CHUNK)], rows.at[b], sem).wait()

    def scat(c, b, sem):
        pltpu.async_copy(rows.at[b], acc.at[dst2_v.at[c]], sem, add=True)

    def scat_wait(b, sem):
        pltpu.make_async_copy(rows.at[b], acc.at[pl.ds(0, CHUNK)], sem).wait()

    def run_edges(tab):
        @pl.loop(0, 2)
        def _(seg):
            pltpu.sync_copy(srcr.at[seg, sid], src_v)
            pltpu.sync_copy(dst2.at[seg, sid], dst2_v)
            gath(tab, 0, 0, sga)

            @pl.loop(0, NCH, step=2)
            def _(c):
                gath(tab, c + 1, 1, sgb)
                gath_wait(0, sga)
                scat(c, 0, ssa)
                scat_wait(0, ssa)

                @pl.when(c + 2 < NCH)
                def _():
                    gath(tab, c + 2, 0, sga)

                gath_wait(1, sgb)
                scat(c + 1, 1, ssb)
                scat_wait(1, ssb)

    _zero_acc(rows, acc, base)
    plsc.subcore_barrier()

    @pl.when(cid == 0)
    def _():
        run_edges(ylo)

    @pl.when(cid == 1)
    def _():
        run_edges(yhi)

    plsc.subcore_barrier()
    pltpu.sync_copy(acc.at[pl.ds(base, RPS)], out.at[cid, pl.ds(base, RPS)])

    if compute_deg:
        _zero_acc(rows, acc, base)
        ones16 = jnp.ones((L,), jnp.float32)

        @pl.loop(0, CHUNK)
        def _(r):
            @pl.loop(0, HALF // L)
            def _(j):
                rows[0, r, pl.ds(j * L, L)] = ones16

        plsc.subcore_barrier()

        @pl.when(cid == 0)
        def _():
            @pl.loop(0, 2)
            def _(seg):
                pltpu.sync_copy(dst2.at[seg, sid], dst2_v)

                @pl.loop(0, NCH, step=4)
                def _(c):
                    for b in range(4):
                        scat(c + b, 0, ssa)
                    for b in range(4):
                        scat_wait(0, ssa)

            plsc.subcore_barrier()
            pltpu.sync_copy(acc.at[pl.ds(base, RPS)], deg.at[pl.ds(base, RPS)])


def _make_sc_agg(compute_deg):
    out_type = jax.ShapeDtypeStruct((NC, NPAD, HALF), jnp.float32)
    if compute_deg:
        out_type = [out_type, jax.ShapeDtypeStruct((NPAD, HALF), jnp.float32)]
    return pl.kernel(
        functools.partial(_sc_agg_body, compute_deg),
        out_type=out_type,
        mesh=_mesh,
        scratch_types=[
            pltpu.VMEM((EWP // 128, 128), jnp.int32),
            pltpu.VMEM((NCH, CHUNK), jnp.int32),
            pltpu.VMEM((2, CHUNK, HALF), jnp.float32),
            pltpu.VMEM_SHARED((NPAD, HALF), jnp.float32),
            pltpu.SemaphoreType.DMA,
            pltpu.SemaphoreType.DMA,
            pltpu.SemaphoreType.DMA,
            pltpu.SemaphoreType.DMA,
        ],
        compiler_params=pltpu.CompilerParams(needs_layout_passes=False),
        name=f"sage_sc_agg_deg{int(compute_deg)}",
    )


_sc_agg_deg = _make_sc_agg(True)
_sc_agg = _make_sc_agg(False)

BN = 2000


def _mm_body(x_ref, ws_ref, wn_ref, ys_ref, ylo_ref, yhi_ref):
    x = x_ref[...]
    ys_ref[...] = jnp.dot(x, ws_ref[...], preferred_element_type=jnp.float32)
    yn = jnp.dot(x, wn_ref[...], preferred_element_type=jnp.float32)
    ylo_ref[...] = yn[:, :HALF]
    yhi_ref[...] = yn[:, HALF:]


_MM_OUT_SPECS = [
    pl.BlockSpec((BN, H), lambda i: (i, 0)),
    pl.BlockSpec((BN, HALF), lambda i: (i, 0)),
    pl.BlockSpec((BN, HALF), lambda i: (i, 0)),
]
_MM_OUT_SHAPE = [
    jax.ShapeDtypeStruct((N, H), jnp.float32),
    jax.ShapeDtypeStruct((N, HALF), jnp.float32),
    jax.ShapeDtypeStruct((N, HALF), jnp.float32),
]

_mm = pl.pallas_call(
    _mm_body,
    grid=(N // BN,),
    in_specs=[
        pl.BlockSpec((BN, D), lambda i: (i, 0)),
        pl.BlockSpec((D, H), lambda i: (0, 0)),
        pl.BlockSpec((D, H), lambda i: (0, 0)),
    ],
    out_specs=_MM_OUT_SPECS,
    out_shape=_MM_OUT_SHAPE,
)


def _comb_core(ys_ref, a_ref, deg_ref, b_ref):
    a = a_ref[...]
    mean = jnp.concatenate([a[0], a[1]], axis=-1)
    recip = 1.0 / jnp.maximum(deg_ref[...], 1.0)
    return ys_ref[...] + mean * recip + b_ref[...]


def _comb_mm_body(with_scale, *refs):
    if with_scale:
        ys_ref, a_ref, deg_ref, b_ref, scale_ref, ws_ref, wn_ref, ys_o, ylo_o, yhi_o = refs
    else:
        ys_ref, a_ref, deg_ref, b_ref, ws_ref, wn_ref, ys_o, ylo_o, yhi_o = refs
    hv = jnp.tanh(_comb_core(ys_ref, a_ref, deg_ref, b_ref))
    if with_scale:
        hv = hv * scale_ref[...]
    ys_o[...] = jnp.dot(hv, ws_ref[...], preferred_element_type=jnp.float32)
    yn = jnp.dot(hv, wn_ref[...], preferred_element_type=jnp.float32)
    ylo_o[...] = yn[:, :HALF]
    yhi_o[...] = yn[:, HALF:]


def _comb_final_body(ys_ref, a_ref, deg_ref, b_ref, out_ref):
    out_ref[...] = _comb_core(ys_ref, a_ref, deg_ref, b_ref)


_COMB_IN_SPECS = [
    pl.BlockSpec((BN, H), lambda i: (i, 0)),
    pl.BlockSpec((NC, BN, HALF), lambda i: (0, i, 0)),
    pl.BlockSpec((BN, 1), lambda i: (i, 0)),
    pl.BlockSpec((H,), lambda i: (0,)),
]


def _make_comb_mm(with_scale):
    extra = [pl.BlockSpec((BN, H), lambda i: (i, 0))] if with_scale else []
    return pl.pallas_call(
        functools.partial(_comb_mm_body, with_scale),
        grid=(N // BN,),
        in_specs=_COMB_IN_SPECS + extra + [
            pl.BlockSpec((H, H), lambda i: (0, 0)),
            pl.BlockSpec((H, H), lambda i: (0, 0)),
        ],
        out_specs=_MM_OUT_SPECS,
        out_shape=_MM_OUT_SHAPE,
    )


_comb_mm_drop = _make_comb_mm(True)
_comb_mm = _make_comb_mm(False)

_comb_final = pl.pallas_call(
    _comb_final_body,
    grid=(N // BN,),
    in_specs=_COMB_IN_SPECS,
    out_specs=pl.BlockSpec((BN, H), lambda i: (i, 0)),
    out_shape=jax.ShapeDtypeStruct((N, H), jnp.float32),
)


def kernel(in_feat, edge_index, Ws1, Wn1, b1, Ws2, Wn2, b2, Ws3, Wn3, b3):
    src = edge_index[0]
    dst = edge_index[1]
    pad = EP - E
    spread = jnp.arange(pad, dtype=jnp.int32)
    src_p = jnp.concatenate([src, spread % N])
    dst_p = jnp.concatenate([dst, N + spread % (NPAD - N)])
    srcr = src_p.reshape(2, NS, EWP // 128, 128)
    dst2 = dst_p.reshape(2, NS, NCH, CHUNK)

    mask = jax.random.bernoulli(jax.random.key(42), 0.5, (N, H))
    scale = mask.astype(jnp.float32) * 2.0

    y1s, y1lo, y1hi = _mm(in_feat, Ws1, Wn1)
    agg1, degp = _sc_agg_deg(y1lo, y1hi, srcr, dst2)
    deg = degp[:N, :1]
    y2s, y2lo, y2hi = _comb_mm_drop(y1s, agg1, deg, b1, scale, Ws2, Wn2)
    agg2 = _sc_agg(y2lo, y2hi, srcr, dst2)
    y3s, y3lo, y3hi = _comb_mm(y2s, agg2, deg, b2, Ws3, Wn3)
    agg3 = _sc_agg(y3lo, y3hi, srcr, dst2)
    return _comb_final(y3s, agg3, deg, b3)
